# Initial kernel scaffold; baseline (speedup 1.0000x reference)
import jax
import jax.numpy as jnp
from jax.experimental import pallas as pl


def kernel(p, x, o, W, gamma, beta):
    raise NotImplementedError("write your pallas kernel here")


# speedup vs baseline: 1.2010x; 1.2010x over previous
import functools

import jax
import jax.numpy as jnp
from jax.experimental import pallas as pl


def _mm_stats_body(x_ref, w_ref, h_ref, stats_ref):
    i = pl.program_id(0)
    h = jax.lax.dot_general(
        x_ref[...], w_ref[...],
        dimension_numbers=(((1,), (1,)), ((), ())),
        preferred_element_type=jnp.float32,
    )
    h_ref[...] = h
    s = jnp.sum(h, axis=0)
    ss = jnp.sum(h * h, axis=0)
    row = jax.lax.broadcasted_iota(jnp.int32, stats_ref.shape, 0)
    contrib = jnp.where(row == 0, s[None, :], 0.0) + jnp.where(row == 1, ss[None, :], 0.0)

    @pl.when(i == 0)
    def _():
        stats_ref[...] = contrib

    @pl.when(i != 0)
    def _():
        stats_ref[...] += contrib


def _norm_body(n_rows, h_ref, stats_ref, gamma_ref, beta_ref, out_ref):
    stats = stats_ref[...]
    mean = stats[0:1, :] / n_rows
    ex2 = stats[1:2, :] / n_rows
    var = ex2 - mean * mean
    inv = jax.lax.rsqrt(var + 1e-5)
    scale = gamma_ref[...] * inv
    bias = beta_ref[...] - mean * scale
    out_ref[...] = jnp.maximum(h_ref[...] * scale + bias, 0.0)


def _pick_tile(n, candidates):
    for c in candidates:
        if n % c == 0 and c % 8 == 0:
            return c
    return n


def kernel(p, x, o, W, gamma, beta):
    n, c_in = x.shape
    c_out = W.shape[0]

    r1 = _pick_tile(n, (2000, 1000, 500 * 8, 8))
    grid1 = n // r1
    h, stats = pl.pallas_call(
        _mm_stats_body,
        grid=(grid1,),
        in_specs=[
            pl.BlockSpec((r1, c_in), lambda i: (i, 0)),
            pl.BlockSpec((c_out, c_in), lambda i: (0, 0)),
        ],
        out_specs=[
            pl.BlockSpec((r1, c_out), lambda i: (i, 0)),
            pl.BlockSpec((8, c_out), lambda i: (0, 0)),
        ],
        out_shape=[
            jax.ShapeDtypeStruct((n, c_out), jnp.float32),
            jax.ShapeDtypeStruct((8, c_out), jnp.float32),
        ],
    )(x, W)

    r2 = _pick_tile(n, (4000, 2000, 1000, 8))
    grid2 = n // r2
    out = pl.pallas_call(
        functools.partial(_norm_body, float(n)),
        grid=(grid2,),
        in_specs=[
            pl.BlockSpec((r2, c_out), lambda i: (i, 0)),
            pl.BlockSpec((8, c_out), lambda i: (0, 0)),
            pl.BlockSpec((1, c_out), lambda i: (0, 0)),
            pl.BlockSpec((1, c_out), lambda i: (0, 0)),
        ],
        out_specs=pl.BlockSpec((r2, c_out), lambda i: (i, 0)),
        out_shape=jax.ShapeDtypeStruct((n, c_out), jnp.float32),
    )(h, stats, gamma.reshape(1, c_out), beta.reshape(1, c_out))

    return (p, out, o)


# speedup vs baseline: 1.4831x; 1.2349x over previous
import functools

import jax
import jax.numpy as jnp
from jax.experimental import pallas as pl


def _mm_stats_body(x_ref, w_ref, h_ref, stats_ref):
    i = pl.program_id(0)
    h = jax.lax.dot_general(
        x_ref[...], w_ref[...],
        dimension_numbers=(((1,), (1,)), ((), ())),
        preferred_element_type=jnp.float32,
    )
    h_ref[...] = h.astype(h_ref.dtype)
    s = jnp.sum(h, axis=0)
    ss = jnp.sum(h * h, axis=0)
    row = jax.lax.broadcasted_iota(jnp.int32, stats_ref.shape, 0)
    contrib = jnp.where(row == 0, s[None, :], 0.0) + jnp.where(row == 1, ss[None, :], 0.0)

    @pl.when(i == 0)
    def _():
        stats_ref[...] = contrib

    @pl.when(i != 0)
    def _():
        stats_ref[...] += contrib


def _norm_body(n_rows, h_ref, stats_ref, gamma_ref, beta_ref, out_ref):
    stats = stats_ref[...]
    mean = stats[0:1, :] / n_rows
    ex2 = stats[1:2, :] / n_rows
    var = ex2 - mean * mean
    inv = jax.lax.rsqrt(var + 1e-5)
    scale = gamma_ref[...] * inv
    bias = beta_ref[...] - mean * scale
    out_ref[...] = jnp.maximum(h_ref[...].astype(jnp.float32) * scale + bias, 0.0)


def _pick_tile(n, candidates):
    for c in candidates:
        if n % c == 0 and c % 8 == 0:
            return c
    return n


def kernel(p, x, o, W, gamma, beta):
    n, c_in = x.shape
    c_out = W.shape[0]

    r1 = _pick_tile(n, (2000, 1000, 500 * 8, 8))
    grid1 = n // r1
    h, stats = pl.pallas_call(
        _mm_stats_body,
        grid=(grid1,),
        in_specs=[
            pl.BlockSpec((r1, c_in), lambda i: (i, 0)),
            pl.BlockSpec((c_out, c_in), lambda i: (0, 0)),
        ],
        out_specs=[
            pl.BlockSpec((r1, c_out), lambda i: (i, 0)),
            pl.BlockSpec((8, c_out), lambda i: (0, 0)),
        ],
        out_shape=[
            jax.ShapeDtypeStruct((n, c_out), jnp.bfloat16),
            jax.ShapeDtypeStruct((8, c_out), jnp.float32),
        ],
    )(x, W)

    r2 = _pick_tile(n, (4000, 2000, 1000, 8))
    grid2 = n // r2
    out = pl.pallas_call(
        functools.partial(_norm_body, float(n)),
        grid=(grid2,),
        in_specs=[
            pl.BlockSpec((r2, c_out), lambda i: (i, 0)),
            pl.BlockSpec((8, c_out), lambda i: (0, 0)),
            pl.BlockSpec((1, c_out), lambda i: (0, 0)),
            pl.BlockSpec((1, c_out), lambda i: (0, 0)),
        ],
        out_specs=pl.BlockSpec((r2, c_out), lambda i: (i, 0)),
        out_shape=jax.ShapeDtypeStruct((n, c_out), jnp.float32),
    )(h, stats, gamma.reshape(1, c_out), beta.reshape(1, c_out))

    return (p, out, o)


# speedup vs baseline: 1.5372x; 1.0365x over previous
import functools

import jax
import jax.numpy as jnp
from jax.experimental import pallas as pl


def _mm_stats_body(x_ref, w_ref, h_ref, stats_ref):
    i = pl.program_id(0)
    h = jax.lax.dot_general(
        x_ref[...], w_ref[...],
        dimension_numbers=(((1,), (1,)), ((), ())),
        preferred_element_type=jnp.float32,
    )
    h_ref[...] = h.astype(h_ref.dtype)
    s = jnp.sum(h, axis=0)
    ss = jnp.sum(h * h, axis=0)
    row = jax.lax.broadcasted_iota(jnp.int32, stats_ref.shape, 0)
    contrib = jnp.where(row == 0, s[None, :], 0.0) + jnp.where(row == 1, ss[None, :], 0.0)

    @pl.when(i == 0)
    def _():
        stats_ref[...] = contrib

    @pl.when(i != 0)
    def _():
        stats_ref[...] += contrib


def _norm_body(n_rows, h_ref, stats_ref, gamma_ref, beta_ref, out_ref):
    stats = stats_ref[...]
    mean = stats[0:1, :] / n_rows
    ex2 = stats[1:2, :] / n_rows
    var = ex2 - mean * mean
    inv = jax.lax.rsqrt(var + 1e-5)
    scale = gamma_ref[...] * inv
    bias = beta_ref[...] - mean * scale
    out_ref[...] = jnp.maximum(h_ref[...].astype(jnp.float32) * scale + bias, 0.0)


def _pick_tile(n, candidates):
    for c in candidates:
        if n % c == 0 and c % 8 == 0:
            return c
    return n


def kernel(p, x, o, W, gamma, beta):
    n, c_in = x.shape
    c_out = W.shape[0]

    r1 = _pick_tile(n, (4000, 2000, 1000, 8))
    grid1 = n // r1
    h, stats = pl.pallas_call(
        _mm_stats_body,
        grid=(grid1,),
        in_specs=[
            pl.BlockSpec((r1, c_in), lambda i: (i, 0)),
            pl.BlockSpec((c_out, c_in), lambda i: (0, 0)),
        ],
        out_specs=[
            pl.BlockSpec((r1, c_out), lambda i: (i, 0)),
            pl.BlockSpec((8, c_out), lambda i: (0, 0)),
        ],
        out_shape=[
            jax.ShapeDtypeStruct((n, c_out), jnp.bfloat16),
            jax.ShapeDtypeStruct((8, c_out), jnp.float32),
        ],
    )(x, W)

    r2 = _pick_tile(n, (5000, 4000, 2000, 1000, 8))
    grid2 = n // r2
    out = pl.pallas_call(
        functools.partial(_norm_body, float(n)),
        grid=(grid2,),
        in_specs=[
            pl.BlockSpec((r2, c_out), lambda i: (i, 0)),
            pl.BlockSpec((8, c_out), lambda i: (0, 0)),
            pl.BlockSpec((1, c_out), lambda i: (0, 0)),
            pl.BlockSpec((1, c_out), lambda i: (0, 0)),
        ],
        out_specs=pl.BlockSpec((r2, c_out), lambda i: (i, 0)),
        out_shape=jax.ShapeDtypeStruct((n, c_out), jnp.float32),
    )(h, stats, gamma.reshape(1, c_out), beta.reshape(1, c_out))

    return (p, out, o)


# speedup vs baseline: 1.5576x; 1.0133x over previous
import functools

import jax
import jax.numpy as jnp
from jax.experimental import pallas as pl


def _mm_stats_body(x_ref, w_ref, h_ref, stats_ref):
    i = pl.program_id(0)
    h = jax.lax.dot_general(
        x_ref[...].astype(jnp.bfloat16), w_ref[...].astype(jnp.bfloat16),
        dimension_numbers=(((1,), (1,)), ((), ())),
        preferred_element_type=jnp.float32,
    )
    h_ref[...] = h.astype(h_ref.dtype)
    s = jnp.sum(h, axis=0)
    ss = jnp.sum(h * h, axis=0)
    row = jax.lax.broadcasted_iota(jnp.int32, stats_ref.shape, 0)
    contrib = jnp.where(row == 0, s[None, :], 0.0) + jnp.where(row == 1, ss[None, :], 0.0)

    @pl.when(i == 0)
    def _():
        stats_ref[...] = contrib

    @pl.when(i != 0)
    def _():
        stats_ref[...] += contrib


def _norm_body(n_rows, h_ref, stats_ref, gamma_ref, beta_ref, out_ref):
    stats = stats_ref[...]
    mean = stats[0:1, :] / n_rows
    ex2 = stats[1:2, :] / n_rows
    var = ex2 - mean * mean
    inv = jax.lax.rsqrt(var + 1e-5)
    scale = gamma_ref[...] * inv
    bias = beta_ref[...] - mean * scale
    out_ref[...] = jnp.maximum(h_ref[...].astype(jnp.float32) * scale + bias, 0.0)


def _pick_tile(n, candidates):
    for c in candidates:
        if n % c == 0 and c % 8 == 0:
            return c
    return n


def kernel(p, x, o, W, gamma, beta):
    n, c_in = x.shape
    c_out = W.shape[0]

    r1 = _pick_tile(n, (4000, 2000, 1000, 8))
    grid1 = n // r1
    h, stats = pl.pallas_call(
        _mm_stats_body,
        grid=(grid1,),
        in_specs=[
            pl.BlockSpec((r1, c_in), lambda i: (i, 0)),
            pl.BlockSpec((c_out, c_in), lambda i: (0, 0)),
        ],
        out_specs=[
            pl.BlockSpec((r1, c_out), lambda i: (i, 0)),
            pl.BlockSpec((8, c_out), lambda i: (0, 0)),
        ],
        out_shape=[
            jax.ShapeDtypeStruct((n, c_out), jnp.bfloat16),
            jax.ShapeDtypeStruct((8, c_out), jnp.float32),
        ],
    )(x, W)

    r2 = _pick_tile(n, (5000, 4000, 2000, 1000, 8))
    grid2 = n // r2
    out = pl.pallas_call(
        functools.partial(_norm_body, float(n)),
        grid=(grid2,),
        in_specs=[
            pl.BlockSpec((r2, c_out), lambda i: (i, 0)),
            pl.BlockSpec((8, c_out), lambda i: (0, 0)),
            pl.BlockSpec((1, c_out), lambda i: (0, 0)),
            pl.BlockSpec((1, c_out), lambda i: (0, 0)),
        ],
        out_specs=pl.BlockSpec((r2, c_out), lambda i: (i, 0)),
        out_shape=jax.ShapeDtypeStruct((n, c_out), jnp.float32),
    )(h, stats, gamma.reshape(1, c_out), beta.reshape(1, c_out))

    return (p, out, o)


# speedup vs baseline: 1.5604x; 1.0018x over previous
import functools

import jax
import jax.numpy as jnp
from jax.experimental import pallas as pl
from jax.experimental.pallas import tpu as pltpu


def _fused_body(n_rows, g, r, x_ref, w_ref, gamma_ref, beta_ref,
                out_ref, h_any, stats, hbuf, ibuf, sem_out, sem_in):
    i = pl.program_id(0)

    @pl.when(i < g)
    def _phase0():
        h = jax.lax.dot_general(
            x_ref[...].astype(jnp.bfloat16), w_ref[...].astype(jnp.bfloat16),
            dimension_numbers=(((1,), (1,)), ((), ())),
            preferred_element_type=jnp.float32,
        )
        s = jnp.sum(h, axis=0)
        ss = jnp.sum(h * h, axis=0)
        row = jax.lax.broadcasted_iota(jnp.int32, stats.shape, 0)
        contrib = (jnp.where(row == 0, s[None, :], 0.0)
                   + jnp.where(row == 1, ss[None, :], 0.0))

        @pl.when(i == 0)
        def _():
            stats[...] = contrib

        @pl.when(i != 0)
        def _():
            stats[...] += contrib

        slot = jax.lax.rem(i, 2)

        @pl.when(i >= 2)
        def _():
            pltpu.make_async_copy(
                hbuf.at[slot], h_any.at[pl.ds(jnp.maximum(i - 2, 0) * r, r)],
                sem_out.at[slot]
            ).wait()

        hbuf[slot] = h.astype(jnp.bfloat16)
        pltpu.make_async_copy(
            hbuf.at[slot], h_any.at[pl.ds(jnp.minimum(i, g - 1) * r, r)],
            sem_out.at[slot]
        ).start()

        @pl.when(i == g - 1)
        def _():
            if g <= 2:
                pltpu.make_async_copy(
                    hbuf.at[0], h_any.at[pl.ds(0, r)], sem_out.at[0]
                ).wait()
            pltpu.make_async_copy(
                h_any.at[pl.ds(0, r)], ibuf.at[0], sem_in.at[0]
            ).start()

    @pl.when(i >= g)
    def _phase1():
        j = jnp.clip(i - g, 0, g - 1)
        slot = jax.lax.rem(j, 2)

        _drain = [k for k in range(max(g - 2, 0), g) if not (g <= 2 and k == 0)]
        if _drain:
            @pl.when(j == 0)
            def _():
                for k in _drain:
                    pltpu.make_async_copy(
                        hbuf.at[k % 2],
                        h_any.at[pl.ds(k * r, r)],
                        sem_out.at[k % 2],
                    ).wait()

        @pl.when(j + 1 < g)
        def _():
            jn = jnp.minimum(j + 1, g - 1)
            nslot = jax.lax.rem(jn, 2)
            pltpu.make_async_copy(
                h_any.at[pl.ds(jn * r, r)], ibuf.at[nslot], sem_in.at[nslot]
            ).start()

        pltpu.make_async_copy(
            h_any.at[pl.ds(j * r, r)], ibuf.at[slot], sem_in.at[slot]
        ).wait()

        st = stats[...]
        mean = st[0:1, :] / n_rows
        ex2 = st[1:2, :] / n_rows
        var = ex2 - mean * mean
        inv = jax.lax.rsqrt(var + 1e-5)
        scale = gamma_ref[...] * inv
        bias = beta_ref[...] - mean * scale
        out_ref[...] = jnp.maximum(
            ibuf[slot].astype(jnp.float32) * scale + bias, 0.0)


def _pick_tile(n, candidates):
    for c in candidates:
        if n % c == 0 and c % 8 == 0:
            return c
    return n


def kernel(p, x, o, W, gamma, beta):
    n, c_in = x.shape
    c_out = W.shape[0]

    r = _pick_tile(n, (4000, 2000, 1000, 8))
    g = n // r
    out, _ = pl.pallas_call(
        functools.partial(_fused_body, float(n), g, r),
        grid=(2 * g,),
        in_specs=[
            pl.BlockSpec((r, c_in), lambda i, g=g: (jnp.where(i < g, i, g - 1), 0)),
            pl.BlockSpec((c_out, c_in), lambda i: (0, 0)),
            pl.BlockSpec((1, c_out), lambda i: (0, 0)),
            pl.BlockSpec((1, c_out), lambda i: (0, 0)),
        ],
        out_specs=[
            pl.BlockSpec((r, c_out), lambda i, g=g: (jnp.where(i < g, 0, i - g), 0)),
            pl.BlockSpec(memory_space=pltpu.MemorySpace.HBM),
        ],
        out_shape=[
            jax.ShapeDtypeStruct((n, c_out), jnp.float32),
            jax.ShapeDtypeStruct((n, c_out), jnp.bfloat16),
        ],
        scratch_shapes=[
            pltpu.VMEM((8, c_out), jnp.float32),
            pltpu.VMEM((2, r, c_out), jnp.bfloat16),
            pltpu.VMEM((2, r, c_out), jnp.bfloat16),
            pltpu.SemaphoreType.DMA((2,)),
            pltpu.SemaphoreType.DMA((2,)),
        ],
    )(x, W, gamma.reshape(1, c_out), beta.reshape(1, c_out))

    return (p, out, o)
